# baseline (speedup 1.0000x reference)
import functools

import jax
import jax.numpy as jnp
from jax import lax
from jax.experimental import pallas as pl
from jax.experimental.pallas import tpu as pltpu
from jax.experimental.pallas import tpu_sc as plsc

S = 512
B = 32
C3 = 3
PLANES = B * C3
RCH = 16
CW = 128
NCHUNK = S // RCH
ITEMS = PLANES * NCHUNK
NWORK = 32
PER_W = ITEMS // NWORK
LANES = 16


APB = 8
AGRID = PLANES // APB
AR = APB * S


def _copy_sum_body(x_ref, out_ref, cs_ref):
    x = x_ref[...]
    out_ref[...] = x
    cs_ref[0] = jnp.sum(x.reshape(APB, S, S), axis=1)


def _pass_a(image2d):
    return pl.pallas_call(
        _copy_sum_body,
        grid=(AGRID,),
        in_specs=[pl.BlockSpec((AR, S), lambda g: (g, 0))],
        out_specs=[
            pl.BlockSpec((AR, S), lambda g: (g, 0)),
            pl.BlockSpec((1, APB, S), lambda g: (g, 0, 0)),
        ],
        out_shape=[
            jax.ShapeDtypeStruct((PLANES * S, S), jnp.float32),
            jax.ShapeDtypeStruct((AGRID, APB, S), jnp.float32),
        ],
    )(image2d)


def _color_body(cs_ref, ann_ref, w_ref, b_ref, fparams_ref, ibounds_ref):
    cs = cs_ref[...]
    sums = jnp.sum(cs[:, 0, 0, :], axis=1)
    pooled = sums.reshape(B, C3) * (1.0 / (S * S))
    feat = jnp.dot(pooled, w_ref[...],
                   preferred_element_type=jnp.float32) + b_ref[...]
    sig = 1.0 / (1.0 + jnp.exp(-feat))
    color = sig[:, :3]
    t = sig[:, 3:4]
    fparams = jnp.concatenate(
        [color * (1.0 - t), t, jnp.zeros((B, 12), jnp.float32)], axis=1)
    fparams_ref[...] = fparams

    ann = jnp.clip(ann_ref[...], 0.0, 1.0) * S
    x1 = ann[:, 0:1]
    y1 = ann[:, 1:2]
    x2 = x1 + ann[:, 2:3]
    y2 = y1 + ann[:, 3:4]
    xlo = jnp.floor(x1) + 1.0
    ylo = jnp.floor(y1) + 1.0
    xhi = jnp.minimum(jnp.ceil(x2) - 1.0, S - 1.0)
    yhi = jnp.minimum(jnp.ceil(y2) - 1.0, S - 1.0)
    bounds = jnp.concatenate([ylo, yhi, xlo, xhi], axis=1).astype(jnp.int32)
    ibounds_ref[...] = jnp.concatenate(
        [bounds, jnp.zeros((B, 12), jnp.int32)], axis=1)


def _pass_c(colsums, annotations, W_color, b_color):
    return pl.pallas_call(
        _color_body,
        in_specs=[pl.BlockSpec((PLANES, 1, 1, S), lambda: (0, 0, 0, 0)),
                  pl.BlockSpec((B, 4), lambda: (0, 0)),
                  pl.BlockSpec((C3, 4), lambda: (0, 0)),
                  pl.BlockSpec((1, 4), lambda: (0, 0))],
        out_specs=[pl.BlockSpec((B, 16), lambda: (0, 0)),
                   pl.BlockSpec((B, 16), lambda: (0, 0))],
        out_shape=[jax.ShapeDtypeStruct((B, 16), jnp.float32),
                   jax.ShapeDtypeStruct((B, 16), jnp.int32)],
    )(colsums, annotations, W_color, b_color.reshape(1, 4))


def _sc_blend(img_ref, fparams_hbm, ibounds_hbm,
              fparams_v, ibounds_v, ch0, ch1, ch2, ch3, ch4, ch5, list_s,
              si0, si1, si2, si3, si4, si5, so0, so1, so2, so3, so4, so5):
    pltpu.sync_copy(fparams_hbm, fparams_v)
    pltpu.sync_copy(ibounds_hbm, ibounds_v)

    chunks = (ch0, ch1, ch2, ch3, ch4, ch5)
    in_sems = (si0, si1, si2, si3, si4, si5)
    out_sems = (so0, so1, so2, so3, so4, so5)

    wid = lax.axis_index("s") * 2 + lax.axis_index("c")

    def decode(item):
        b = item // (C3 * NCHUNK)
        rem = item % (C3 * NCHUNK)
        c = rem // NCHUNK
        r0 = (rem % NCHUNK) * RCH
        return b, c, r0

    def compact(i, cnt):
        item = wid + NWORK * i
        b, c, r0 = decode(item)
        iv = ibounds_v[b, pl.ds(0, LANES)]
        rlo = jnp.maximum(r0, iv[0])
        rhi = jnp.minimum(r0 + RCH - 1, iv[1])
        active = jnp.logical_and(rlo <= rhi, iv[2] <= iv[3])

        @pl.when(active)
        def _():
            list_s[cnt] = item

        return cnt + active.astype(jnp.int32)

    n = lax.fori_loop(0, PER_W, compact, jnp.int32(0))

    def issue_in(j, slot):
        b, c, r0 = decode(list_s[j])
        pltpu.make_async_copy(
            img_ref.at[b, c, pl.ds(r0, RCH), :], chunks[slot],
            in_sems[slot]).start()

    def wait_in(slot):
        pltpu.make_async_copy(
            img_ref.at[0, 0, pl.ds(0, RCH), :], chunks[slot],
            in_sems[slot]).wait()

    def issue_out(j, slot):
        b, c, r0 = decode(list_s[j])
        pltpu.make_async_copy(
            chunks[slot], img_ref.at[b, c, pl.ds(r0, RCH), :],
            out_sems[slot]).start()

    def wait_out(slot):
        pltpu.make_async_copy(
            chunks[slot], img_ref.at[0, 0, pl.ds(0, RCH), :],
            out_sems[slot]).wait()

    def compute(j, slot):
        b, c, r0 = decode(list_s[j])
        iv = ibounds_v[b, pl.ds(0, LANES)]
        xlo = iv[2]
        xhi = iv[3]
        rlo = jnp.maximum(r0, iv[0])
        rhi = jnp.minimum(r0 + RCH - 1, iv[1])
        fv = fparams_v[b, pl.ds(0, LANES)]
        cval = jnp.where(c == 0, fv[0], jnp.where(c == 1, fv[1], fv[2]))
        t = fv[3]
        chunk = chunks[slot]

        full = jnp.logical_and(rlo == r0, rhi == r0 + RCH - 1)

        def per_colblock(jb, _):
            j0 = jb * LANES
            jv = lax.iota(jnp.int32, LANES) + j0
            cmask = jnp.logical_and(jv >= xlo, jv <= xhi)

            @pl.when(full)
            def _():
                for ri in range(RCH):
                    v = chunk[ri, pl.ds(j0, LANES)]
                    chunk[ri, pl.ds(j0, LANES)] = jnp.where(
                        cmask, cval + t * v, v)

            @pl.when(jnp.logical_not(full))
            def _():
                def per_row(r, _):
                    ri = r - r0
                    v = chunk[ri, pl.ds(j0, LANES)]
                    chunk[ri, pl.ds(j0, LANES)] = jnp.where(
                        cmask, cval + t * v, v)
                    return 0

                lax.fori_loop(rlo, rhi + 1, per_row, 0)

            return 0

        lax.fori_loop(xlo // LANES, xhi // LANES + 1, per_colblock, 0)

    NS_ = 6
    LOOK = 3

    for jj in range(LOOK):
        @pl.when(n > jj)
        def _(jj=jj):
            issue_in(jj, jj)

    def ring_step(i6, _):
        for u in range(NS_):
            j = NS_ * i6 + u

            @pl.when(j < n)
            def _(j=j, u=u):
                wait_in(u)
                compute(j, u)
                issue_out(j, u)
                nslot = (u + LOOK) % NS_

                @pl.when(j + LOOK < n)
                def _():
                    @pl.when(j >= NS_ - LOOK)
                    def _():
                        wait_out(nslot)

                    issue_in(j + LOOK, nslot)

        return 0

    lax.fori_loop(0, (PER_W + NS_ - 1) // NS_, ring_step, 0)

    for d in range(1, NS_ + 1):
        jj = n - d
        for u in range(NS_):
            @pl.when(jnp.logical_and(jj >= 0, jj % NS_ == u))
            def _(u=u):
                wait_out(u)


def _make_sc_kernel():
    mesh = plsc.VectorSubcoreMesh(
        core_axis_name="c", subcore_axis_name="s",
        num_cores=2, num_subcores=16)
    return pl.kernel(
        _sc_blend,
        out_type=(),
        mesh=mesh,
        scratch_types=(
            [pltpu.VMEM((B, 16), jnp.float32),
             pltpu.VMEM((B, 16), jnp.int32)]
            + [pltpu.VMEM((RCH, S), jnp.float32) for _ in range(6)]
            + [pltpu.SMEM((PER_W,), jnp.int32)]
            + [pltpu.SemaphoreType.DMA for _ in range(12)]
        ),
    )


def kernel(image, annotations, W_color, b_color):
    copy2d, colsums = _pass_a(image.reshape(PLANES * S, S))
    copy = copy2d.reshape(B, C3, S, S)
    fparams, ibounds = _pass_c(
        colsums.reshape(PLANES, 1, 1, S), annotations, W_color, b_color)
    ref = jax.new_ref(copy)
    _make_sc_kernel()(ref, fparams, ibounds)
    return ref[...]


# speedup vs baseline: 1.0366x; 1.0366x over previous
import functools

import jax
import jax.numpy as jnp
from jax import lax
from jax.experimental import pallas as pl
from jax.experimental.pallas import tpu as pltpu
from jax.experimental.pallas import tpu_sc as plsc

S = 512
B = 32
C3 = 3
PLANES = B * C3
RCH = 32
CW = 128
NCHUNK = S // RCH
ITEMS = PLANES * NCHUNK
NWORK = 32
PER_W = ITEMS // NWORK
LANES = 16


APB = 12
AGRID = PLANES // APB
AR = APB * S


def _copy_sum_body(x_ref, out_ref, cs_ref):
    x = x_ref[...]
    out_ref[...] = x
    cs_ref[0] = jnp.sum(x.reshape(APB, S, S), axis=1)


def _pass_a(image2d):
    return pl.pallas_call(
        _copy_sum_body,
        grid=(AGRID,),
        in_specs=[pl.BlockSpec((AR, S), lambda g: (g, 0))],
        out_specs=[
            pl.BlockSpec((AR, S), lambda g: (g, 0)),
            pl.BlockSpec((1, APB, S), lambda g: (g, 0, 0)),
        ],
        out_shape=[
            jax.ShapeDtypeStruct((PLANES * S, S), jnp.float32),
            jax.ShapeDtypeStruct((AGRID, APB, S), jnp.float32),
        ],
    )(image2d)


def _color_body(cs_ref, ann_ref, w_ref, b_ref, fparams_ref, ibounds_ref):
    cs = cs_ref[...]
    sums = jnp.sum(cs[:, 0, 0, :], axis=1)
    pooled = sums.reshape(B, C3) * (1.0 / (S * S))
    feat = jnp.dot(pooled, w_ref[...],
                   preferred_element_type=jnp.float32) + b_ref[...]
    sig = 1.0 / (1.0 + jnp.exp(-feat))
    color = sig[:, :3]
    t = sig[:, 3:4]
    fparams = jnp.concatenate(
        [color * (1.0 - t), t, jnp.zeros((B, 12), jnp.float32)], axis=1)
    fparams_ref[...] = fparams

    ann = jnp.clip(ann_ref[...], 0.0, 1.0) * S
    x1 = ann[:, 0:1]
    y1 = ann[:, 1:2]
    x2 = x1 + ann[:, 2:3]
    y2 = y1 + ann[:, 3:4]
    xlo = jnp.floor(x1) + 1.0
    ylo = jnp.floor(y1) + 1.0
    xhi = jnp.minimum(jnp.ceil(x2) - 1.0, S - 1.0)
    yhi = jnp.minimum(jnp.ceil(y2) - 1.0, S - 1.0)
    bounds = jnp.concatenate([ylo, yhi, xlo, xhi], axis=1).astype(jnp.int32)
    ibounds_ref[...] = jnp.concatenate(
        [bounds, jnp.zeros((B, 12), jnp.int32)], axis=1)


def _pass_c(colsums, annotations, W_color, b_color):
    return pl.pallas_call(
        _color_body,
        in_specs=[pl.BlockSpec((PLANES, 1, 1, S), lambda: (0, 0, 0, 0)),
                  pl.BlockSpec((B, 4), lambda: (0, 0)),
                  pl.BlockSpec((C3, 4), lambda: (0, 0)),
                  pl.BlockSpec((1, 4), lambda: (0, 0))],
        out_specs=[pl.BlockSpec((B, 16), lambda: (0, 0)),
                   pl.BlockSpec((B, 16), lambda: (0, 0))],
        out_shape=[jax.ShapeDtypeStruct((B, 16), jnp.float32),
                   jax.ShapeDtypeStruct((B, 16), jnp.int32)],
    )(colsums, annotations, W_color, b_color.reshape(1, 4))


def _sc_blend(img_ref, fparams_hbm, ibounds_hbm,
              fparams_v, ibounds_v, ch0, ch1, ch2, ch3, ch4, ch5, list_s,
              si0, si1, si2, si3, si4, si5, so0, so1, so2, so3, so4, so5):
    pltpu.sync_copy(fparams_hbm, fparams_v)
    pltpu.sync_copy(ibounds_hbm, ibounds_v)

    chunks = (ch0, ch1, ch2, ch3, ch4, ch5)
    in_sems = (si0, si1, si2, si3, si4, si5)
    out_sems = (so0, so1, so2, so3, so4, so5)

    wid = lax.axis_index("s") * 2 + lax.axis_index("c")

    def decode(item):
        b = item // (C3 * NCHUNK)
        rem = item % (C3 * NCHUNK)
        c = rem // NCHUNK
        r0 = (rem % NCHUNK) * RCH
        return b, c, r0

    def compact(i, cnt):
        item = wid + NWORK * i
        b, c, r0 = decode(item)
        iv = ibounds_v[b, pl.ds(0, LANES)]
        rlo = jnp.maximum(r0, iv[0])
        rhi = jnp.minimum(r0 + RCH - 1, iv[1])
        active = jnp.logical_and(rlo <= rhi, iv[2] <= iv[3])

        @pl.when(active)
        def _():
            list_s[cnt] = item

        return cnt + active.astype(jnp.int32)

    n = lax.fori_loop(0, PER_W, compact, jnp.int32(0))

    def issue_in(j, slot):
        b, c, r0 = decode(list_s[j])
        pltpu.make_async_copy(
            img_ref.at[b, c, pl.ds(r0, RCH), :], chunks[slot],
            in_sems[slot]).start()

    def wait_in(slot):
        pltpu.make_async_copy(
            img_ref.at[0, 0, pl.ds(0, RCH), :], chunks[slot],
            in_sems[slot]).wait()

    def issue_out(j, slot):
        b, c, r0 = decode(list_s[j])
        pltpu.make_async_copy(
            chunks[slot], img_ref.at[b, c, pl.ds(r0, RCH), :],
            out_sems[slot]).start()

    def wait_out(slot):
        pltpu.make_async_copy(
            chunks[slot], img_ref.at[0, 0, pl.ds(0, RCH), :],
            out_sems[slot]).wait()

    def compute(j, slot):
        b, c, r0 = decode(list_s[j])
        iv = ibounds_v[b, pl.ds(0, LANES)]
        xlo = iv[2]
        xhi = iv[3]
        rlo = jnp.maximum(r0, iv[0])
        rhi = jnp.minimum(r0 + RCH - 1, iv[1])
        fv = fparams_v[b, pl.ds(0, LANES)]
        cval = jnp.where(c == 0, fv[0], jnp.where(c == 1, fv[1], fv[2]))
        t = fv[3]
        chunk = chunks[slot]

        full = jnp.logical_and(rlo == r0, rhi == r0 + RCH - 1)

        def per_colblock(jb, _):
            j0 = jb * LANES
            jv = lax.iota(jnp.int32, LANES) + j0
            cmask = jnp.logical_and(jv >= xlo, jv <= xhi)

            @pl.when(full)
            def _():
                for ri in range(RCH):
                    v = chunk[ri, pl.ds(j0, LANES)]
                    chunk[ri, pl.ds(j0, LANES)] = jnp.where(
                        cmask, cval + t * v, v)

            @pl.when(jnp.logical_not(full))
            def _():
                def per_row(r, _):
                    ri = r - r0
                    v = chunk[ri, pl.ds(j0, LANES)]
                    chunk[ri, pl.ds(j0, LANES)] = jnp.where(
                        cmask, cval + t * v, v)
                    return 0

                lax.fori_loop(rlo, rhi + 1, per_row, 0)

            return 0

        lax.fori_loop(xlo // LANES, xhi // LANES + 1, per_colblock, 0)

    NS_ = 6
    LOOK = 3

    for jj in range(LOOK):
        @pl.when(n > jj)
        def _(jj=jj):
            issue_in(jj, jj)

    def ring_step(i6, _):
        for u in range(NS_):
            j = NS_ * i6 + u

            @pl.when(j < n)
            def _(j=j, u=u):
                wait_in(u)
                compute(j, u)
                issue_out(j, u)
                nslot = (u + LOOK) % NS_

                @pl.when(j + LOOK < n)
                def _():
                    @pl.when(j >= NS_ - LOOK)
                    def _():
                        wait_out(nslot)

                    issue_in(j + LOOK, nslot)

        return 0

    lax.fori_loop(0, (PER_W + NS_ - 1) // NS_, ring_step, 0)

    for d in range(1, NS_ + 1):
        jj = n - d
        for u in range(NS_):
            @pl.when(jnp.logical_and(jj >= 0, jj % NS_ == u))
            def _(u=u):
                wait_out(u)


def _make_sc_kernel():
    mesh = plsc.VectorSubcoreMesh(
        core_axis_name="c", subcore_axis_name="s",
        num_cores=2, num_subcores=16)
    return pl.kernel(
        _sc_blend,
        out_type=(),
        mesh=mesh,
        scratch_types=(
            [pltpu.VMEM((B, 16), jnp.float32),
             pltpu.VMEM((B, 16), jnp.int32)]
            + [pltpu.VMEM((RCH, S), jnp.float32) for _ in range(6)]
            + [pltpu.SMEM((PER_W,), jnp.int32)]
            + [pltpu.SemaphoreType.DMA for _ in range(12)]
        ),
    )


def kernel(image, annotations, W_color, b_color):
    copy2d, colsums = _pass_a(image.reshape(PLANES * S, S))
    copy = copy2d.reshape(B, C3, S, S)
    fparams, ibounds = _pass_c(
        colsums.reshape(PLANES, 1, 1, S), annotations, W_color, b_color)
    ref = jax.new_ref(copy)
    _make_sc_kernel()(ref, fparams, ibounds)
    return ref[...]
